# baseline (speedup 1.0000x reference)
import functools

import jax
import jax.numpy as jnp
from jax import lax
from jax.experimental import pallas as pl
from jax.experimental.pallas import tpu as pltpu
from jax.experimental.pallas import tpu_sc as plsc


def kernel(indices, table):
    B = indices.shape[0]
    V, D = table.shape
    info = plsc.get_sparse_core_info()
    NC, NS = info.num_cores, info.num_subcores
    NW = NC * NS
    d_per_w = D // NW
    CHUNK = 8192
    n_chunks = B // CHUNK

    tableT = jnp.transpose(table)

    mesh = plsc.VectorSubcoreMesh(core_axis_name="c", subcore_axis_name="s")

    @functools.partial(
        pl.kernel,
        mesh=mesh,
        compiler_params=pltpu.CompilerParams(needs_layout_passes=False),
        out_type=jax.ShapeDtypeStruct((D, B), jnp.float32),
        scratch_types=[
            pltpu.VMEM((B,), jnp.int32),
            pltpu.VMEM((V,), jnp.float32),
            pltpu.VMEM((CHUNK,), jnp.float32),
        ],
    )
    def gather_kernel(idx_hbm, tab_hbm, out_hbm, idx_v, col_v, out_v):
        wid = lax.axis_index("s") * NC + lax.axis_index("c")
        pltpu.sync_copy(idx_hbm, idx_v)
        for f in range(d_per_w):
            d = wid * d_per_w + f
            pltpu.sync_copy(tab_hbm.at[d], col_v)
            for k in range(n_chunks):

                @plsc.parallel_loop(0, CHUNK // 16, unroll=32)
                def body(i):
                    idx16 = idx_v[pl.ds(k * CHUNK + i * 16, 16)]
                    out_v[pl.ds(i * 16, 16)] = plsc.load_gather(
                        col_v, [idx16]
                    )

                pltpu.sync_copy(out_v, out_hbm.at[d, pl.ds(k * CHUNK, CHUNK)])

    outT = gather_kernel(indices, tableT)
    return jnp.transpose(outT)


# speedup vs baseline: 1.0228x; 1.0228x over previous
import functools

import jax
import jax.numpy as jnp
from jax import lax
from jax.experimental import pallas as pl
from jax.experimental.pallas import tpu as pltpu
from jax.experimental.pallas import tpu_sc as plsc


def kernel(indices, table):
    B = indices.shape[0]
    V, D = table.shape
    info = plsc.get_sparse_core_info()
    NC, NS = info.num_cores, info.num_subcores
    NW = NC * NS
    d_per_w = D // NW
    CHUNK = 8192
    n_chunks = B // CHUNK

    tableT = jnp.transpose(table)

    mesh = plsc.VectorSubcoreMesh(core_axis_name="c", subcore_axis_name="s")

    @functools.partial(
        pl.kernel,
        mesh=mesh,
        compiler_params=pltpu.CompilerParams(needs_layout_passes=False),
        out_type=jax.ShapeDtypeStruct((D, B), jnp.float32),
        scratch_types=[
            pltpu.VMEM((B,), jnp.int32),
            pltpu.VMEM((V,), jnp.float32),
            pltpu.VMEM((CHUNK,), jnp.float32),
        ],
    )
    def gather_kernel(idx_hbm, tab_hbm, out_hbm, idx_v, col_v, out_v):
        wid = lax.axis_index("s") * NC + lax.axis_index("c")
        pltpu.sync_copy(idx_hbm, idx_v)
        for f in range(d_per_w):
            d = wid * d_per_w + f
            pltpu.sync_copy(tab_hbm.at[d], col_v)
            for k in range(n_chunks):

                @plsc.parallel_loop(0, CHUNK // 16, unroll=8)
                def body(i):
                    idx16 = idx_v[pl.ds(k * CHUNK + i * 16, 16)]
                    out_v[pl.ds(i * 16, 16)] = plsc.load_gather(
                        col_v, [idx16]
                    )

                pltpu.sync_copy(out_v, out_hbm.at[d, pl.ds(k * CHUNK, CHUNK)])

    outT = gather_kernel(indices, tableT)
    return jnp.transpose(outT)
